# Initial kernel scaffold; baseline (speedup 1.0000x reference)
import jax
import jax.numpy as jnp
from jax.experimental import pallas as pl


def kernel(x_src, x_dst, edge_index, edge_attr, W_src, W_dst, b_dst, w1, b1, w2, b2, gamma, beta):
    raise NotImplementedError("write your pallas kernel here")


# speedup vs baseline: 2.2806x; 2.2806x over previous
import functools
import math

import jax
import jax.numpy as jnp
from jax import lax
from jax.experimental import pallas as pl
from jax.experimental.pallas import tpu as pltpu
from jax.experimental.pallas import tpu_sc as plsc

N = 10000
E = 160000
D = 256
ED = 16
DH = D // 2
W = 144
N_PAD = 10240
N_ACC = 10016
E_PAD = 163840
EPT = E_PAD // 16
CH = 128
NCHUNK = EPT // CH
ROWS_PER_TILE = N_ACC // 16

_SQRT2 = math.sqrt(2.0)


def _gelu(x):
    return 0.5 * x * (1.0 + lax.erf(x / _SQRT2))


def _table_body(x_ref, w_ref, out_ref):
    y = lax.dot_general(x_ref[...], w_ref[...], (((1,), (1,)), ((), ())),
                        preferred_element_type=jnp.float32)
    b = x_ref.shape[0]
    deg_pad = jnp.concatenate(
        [jnp.ones((b, 1), jnp.float32), jnp.zeros((b, W - DH - 1), jnp.float32)],
        axis=1)
    out_ref[0] = jnp.concatenate([y[:, :DH], deg_pad], axis=1)
    out_ref[1] = jnp.concatenate([y[:, DH:], deg_pad], axis=1)


def _build_table(x_src, W_src):
    BN = 400
    grid = N // BN
    out = pl.pallas_call(
        _table_body,
        grid=(grid,),
        in_specs=[
            pl.BlockSpec((BN, D), lambda i: (i, 0)),
            pl.BlockSpec((D, D), lambda i: (0, 0)),
        ],
        out_specs=pl.BlockSpec((2, BN, W), lambda i: (0, i, 0)),
        out_shape=jax.ShapeDtypeStruct((2, N, W), jnp.float32),
    )(x_src, W_src)
    return out.reshape(2 * N, W)


def _gates(edge_attr_pad, w1, w2, b1, b2):
    BE = 2048
    grid = E_PAD // BE

    def body(e_ref, w1_ref, b1_ref, w2_ref, b2_ref, out_ref):
        h = lax.dot_general(e_ref[...], w1_ref[...], (((1,), (1,)), ((), ())),
                            preferred_element_type=jnp.float32)
        h = _gelu(h + b1_ref[...])
        g = jnp.sum(h * w2_ref[...], axis=1, keepdims=True)
        g = g + b2_ref[0]
        out_ref[...] = jax.nn.sigmoid(g).reshape(1, 8, BE // 8)

    out = pl.pallas_call(
        body,
        grid=(grid,),
        in_specs=[
            pl.BlockSpec((BE, ED), lambda i: (i, 0)),
            pl.BlockSpec((D, ED), lambda i: (0, 0)),
            pl.BlockSpec((D,), lambda i: (0,)),
            pl.BlockSpec((1, D), lambda i: (0, 0)),
            pl.BlockSpec((1,), lambda i: (0,)),
        ],
        out_specs=pl.BlockSpec((1, 8, BE // 8), lambda i: (i, 0, 0)),
        out_shape=jax.ShapeDtypeStruct((grid, 8, BE // 8), jnp.float32),
    )(edge_attr_pad, w1, b1, w2, b2)
    return out.reshape(E_PAD)


def _sc_scatter(table, src_pad, dst_pad, g_pad):
    mesh = plsc.VectorSubcoreMesh(core_axis_name="c", subcore_axis_name="s")

    @functools.partial(
        pl.kernel,
        mesh=mesh,
        out_type=jax.ShapeDtypeStruct((2, N_PAD, W), jnp.float32),
        scratch_types=[
            pltpu.VMEM((CH, W), jnp.float32),
            pltpu.VMEM((CH,), jnp.int32),
            pltpu.VMEM((CH,), jnp.int32),
            pltpu.VMEM((CH,), jnp.float32),
            pltpu.VMEM_SHARED((N_ACC, W), jnp.float32),
            pltpu.SemaphoreType.DMA,
        ],
        compiler_params=pltpu.CompilerParams(use_tc_tiling_on_sc=False),
    )
    def k(table_hbm, src_hbm, dst_hbm, g_hbm, out_hbm,
          rows_v, gidx_v, sidx_v, g_v, acc, sem):
        cid = lax.axis_index("c")
        sid = lax.axis_index("s")
        base = sid * EPT
        off = cid * N

        def zrow(i, _):
            r = i // (W // 16)
            c = (i % (W // 16)) * 16
            rows_v[r, pl.ds(c, 16)] = jnp.zeros((16,), jnp.float32)
            return 0
        lax.fori_loop(0, CH * (W // 16), zrow, 0)
        zbase = sid * ROWS_PER_TILE
        for j in range(ROWS_PER_TILE // CH):
            pltpu.sync_copy(rows_v, acc.at[pl.ds(zbase + j * CH, CH)])
        rem = ROWS_PER_TILE % CH
        if rem:
            pltpu.sync_copy(
                rows_v.at[pl.ds(0, rem)],
                acc.at[pl.ds(zbase + (ROWS_PER_TILE // CH) * CH, rem)])
        plsc.subcore_barrier()

        ii = lax.iota(jnp.int32, 16)
        deg_vec = jnp.where(ii == 0, 1.0, 0.0).astype(jnp.float32)

        def chunk(ci, _):
            ebase = base + ci * CH
            pltpu.sync_copy(src_hbm.at[pl.ds(ebase, CH)], gidx_v)
            pltpu.sync_copy(dst_hbm.at[pl.ds(ebase, CH)], sidx_v)
            pltpu.sync_copy(g_hbm.at[pl.ds(ebase, CH)], g_v)

            def adj(i, _):
                sl = pl.ds(i * 16, 16)
                gidx_v[sl] = gidx_v[sl] + off
                return 0
            lax.fori_loop(0, CH // 16, adj, 0)

            pltpu.async_copy(table_hbm.at[gidx_v], rows_v, sem).wait()

            def edge(e, _):
                g16 = g_v[pl.ds((e // 16) * 16, 16)]
                lane = jnp.full((16,), e % 16, jnp.int32)
                gb = lax.gather(
                    g16, lane[:, None],
                    lax.GatherDimensionNumbers(
                        offset_dims=(), collapsed_slice_dims=(0,),
                        start_index_map=(0,)),
                    (1,),
                    mode=lax.GatherScatterMode.PROMISE_IN_BOUNDS)
                for kk in range(DH // 16):
                    sl = pl.ds(kk * 16, 16)
                    rows_v[e, sl] = rows_v[e, sl] * gb
                rows_v[e, pl.ds(DH, 16)] = deg_vec
                return 0
            lax.fori_loop(0, CH, edge, 0)

            pltpu.sync_copy(rows_v, acc.at[sidx_v], add=True)
            return 0
        lax.fori_loop(0, NCHUNK, chunk, 0)

        plsc.subcore_barrier()
        pltpu.sync_copy(acc.at[pl.ds(zbase, ROWS_PER_TILE)],
                        out_hbm.at[cid, pl.ds(zbase, ROWS_PER_TILE)])

    return k(table, src_pad, dst_pad, g_pad)


def _epilogue(acc, x_dst_pad, W_dst, b_dst, gamma, beta):
    BN = 512
    grid = N_PAD // BN

    def body(a_ref, xd_ref, wd_ref, bd_ref, gm_ref, bt_ref, out_ref):
        lo = a_ref[0, :, :DH]
        hi = a_ref[1, :, :DH]
        sums = jnp.concatenate([lo, hi], axis=1)
        deg = a_ref[0, :, DH:DH + 1]
        degc = jnp.maximum(deg, 1.0)
        z = lax.dot_general(xd_ref[...], wd_ref[...], (((1,), (1,)), ((), ())),
                            preferred_element_type=jnp.float32)
        pre = sums / degc + z + bd_ref[...]
        mu = jnp.mean(pre, axis=1, keepdims=True)
        var = jnp.mean(jnp.square(pre - mu), axis=1, keepdims=True)
        ln = (pre - mu) * lax.rsqrt(var + 1e-5) * gm_ref[...] + bt_ref[...]
        out_ref[...] = _gelu(ln)

    out = pl.pallas_call(
        body,
        grid=(grid,),
        in_specs=[
            pl.BlockSpec((2, BN, W), lambda i: (0, i, 0)),
            pl.BlockSpec((BN, D), lambda i: (i, 0)),
            pl.BlockSpec((D, D), lambda i: (0, 0)),
            pl.BlockSpec((D,), lambda i: (0,)),
            pl.BlockSpec((D,), lambda i: (0,)),
            pl.BlockSpec((D,), lambda i: (0,)),
        ],
        out_specs=pl.BlockSpec((BN, D), lambda i: (i, 0)),
        out_shape=jax.ShapeDtypeStruct((N_PAD, D), jnp.float32),
    )(acc, x_dst_pad, W_dst, b_dst, gamma, beta)
    return out[:N]


def kernel(x_src, x_dst, edge_index, edge_attr, W_src, W_dst, b_dst,
           w1, b1, w2, b2, gamma, beta):
    src = edge_index[0]
    dst = edge_index[1]
    pad = E_PAD - E
    src_pad = jnp.concatenate([src, jnp.zeros((pad,), jnp.int32)])
    dst_pad = jnp.concatenate([dst, jnp.full((pad,), N, jnp.int32)])
    edge_attr_pad = jnp.concatenate(
        [edge_attr, jnp.zeros((pad, ED), jnp.float32)], axis=0)
    x_dst_pad = jnp.concatenate(
        [x_dst, jnp.zeros((N_PAD - N, D), jnp.float32)], axis=0)

    table = _build_table(x_src, W_src)
    g_pad = _gates(edge_attr_pad, w1, w2, b1, b2)
    acc = _sc_scatter(table, src_pad, dst_pad, g_pad)
    return _epilogue(acc, x_dst_pad, W_dst, b_dst, gamma, beta)


# speedup vs baseline: 3.2411x; 1.4212x over previous
import functools
import math

import jax
import jax.numpy as jnp
from jax import lax
from jax.experimental import pallas as pl
from jax.experimental.pallas import tpu as pltpu
from jax.experimental.pallas import tpu_sc as plsc

N = 10000
E = 160000
D = 256
ED = 16
DH = D // 2
W = 144
N_PAD = 10240
N_ACC = 10016
E_PAD = 163840
EPT = E_PAD // 16
CH = 128
NCHUNK = EPT // CH
NPAIR = NCHUNK // 2
MW = 3 * CH
ROWS_PER_TILE = N_ACC // 16

_SQRT2 = math.sqrt(2.0)


def _gelu(x):
    return 0.5 * x * (1.0 + lax.erf(x / _SQRT2))


def _table_body(x_ref, w_ref, out_ref):
    y = lax.dot_general(x_ref[...], w_ref[...], (((1,), (1,)), ((), ())),
                        preferred_element_type=jnp.float32)
    b = x_ref.shape[0]
    deg_pad = jnp.concatenate(
        [jnp.ones((b, 1), jnp.float32), jnp.zeros((b, W - DH - 1), jnp.float32)],
        axis=1)
    out_ref[0] = jnp.concatenate([y[:, :DH], deg_pad], axis=1)
    out_ref[1] = jnp.concatenate([y[:, DH:], deg_pad], axis=1)


def _build_table(x_src, W_src):
    BN = 400
    grid = N // BN
    out = pl.pallas_call(
        _table_body,
        grid=(grid,),
        in_specs=[
            pl.BlockSpec((BN, D), lambda i: (i, 0)),
            pl.BlockSpec((D, D), lambda i: (0, 0)),
        ],
        out_specs=pl.BlockSpec((2, BN, W), lambda i: (0, i, 0)),
        out_shape=jax.ShapeDtypeStruct((2, N, W), jnp.float32),
    )(x_src, W_src)
    return out.reshape(2 * N, W)


def _gates(edge_attr, w1, w2, b1, b2):
    BE = 1600
    grid = E // BE

    def body(e_ref, w1_ref, b1_ref, w2_ref, b2_ref, out_ref):
        h = lax.dot_general(e_ref[...], w1_ref[...], (((1,), (1,)), ((), ())),
                            preferred_element_type=jnp.float32)
        h = _gelu(h + b1_ref[...])
        g = jnp.sum(h * w2_ref[...], axis=1, keepdims=True)
        g = g + b2_ref[0]
        out_ref[...] = jax.nn.sigmoid(g).reshape(1, 8, BE // 8)

    out = pl.pallas_call(
        body,
        grid=(grid,),
        in_specs=[
            pl.BlockSpec((BE, ED), lambda i: (i, 0)),
            pl.BlockSpec((D, ED), lambda i: (0, 0)),
            pl.BlockSpec((D,), lambda i: (0,)),
            pl.BlockSpec((1, D), lambda i: (0, 0)),
            pl.BlockSpec((1,), lambda i: (0,)),
        ],
        out_specs=pl.BlockSpec((1, 8, BE // 8), lambda i: (i, 0, 0)),
        out_shape=jax.ShapeDtypeStruct((grid, 8, BE // 8), jnp.float32),
    )(edge_attr, w1, b1, w2, b2)
    return out.reshape(E)


def _sc_scatter(table, meta):
    mesh = plsc.VectorSubcoreMesh(core_axis_name="c", subcore_axis_name="s")

    @functools.partial(
        pl.kernel,
        mesh=mesh,
        out_type=jax.ShapeDtypeStruct((2, N_PAD, W), jnp.float32),
        scratch_types=[
            pltpu.VMEM((CH, W), jnp.float32),
            pltpu.VMEM((CH, W), jnp.float32),
            pltpu.VMEM((MW,), jnp.int32),
            pltpu.VMEM((MW,), jnp.int32),
            pltpu.VMEM((CH,), jnp.int32),
            pltpu.VMEM((CH,), jnp.int32),
            pltpu.VMEM_SHARED((N_ACC, W), jnp.float32),
            pltpu.SemaphoreType.DMA,
            pltpu.SemaphoreType.DMA,
            pltpu.SemaphoreType.DMA,
            pltpu.SemaphoreType.DMA,
            pltpu.SemaphoreType.DMA,
            pltpu.SemaphoreType.DMA,
        ],
        compiler_params=pltpu.CompilerParams(use_tc_tiling_on_sc=False,
                                             needs_layout_passes=False),
    )
    def k(table_hbm, meta_hbm, out_hbm,
          rows0, rows1, meta0, meta1, sidx0, sidx1, acc,
          gsem0, gsem1, msem0, msem1, ssem0, ssem1):
        cid = lax.axis_index("c")
        sid = lax.axis_index("s")
        rowbase = sid * NCHUNK

        def zrow(i, _):
            r = i // (W // 16)
            c = (i % (W // 16)) * 16
            rows0[r, pl.ds(c, 16)] = jnp.zeros((16,), jnp.float32)
            return 0
        lax.fori_loop(0, CH * (W // 16), zrow, 0)
        zbase = sid * ROWS_PER_TILE
        for j in range(ROWS_PER_TILE // CH):
            pltpu.sync_copy(rows0, acc.at[pl.ds(zbase + j * CH, CH)])
        rem = ROWS_PER_TILE % CH
        if rem:
            pltpu.sync_copy(
                rows0.at[pl.ds(0, rem)],
                acc.at[pl.ds(zbase + (ROWS_PER_TILE // CH) * CH, rem)])
        plsc.subcore_barrier()

        ii = lax.iota(jnp.int32, 16)
        deg_vec = jnp.where(ii == 0, 1.0, 0.0).astype(jnp.float32)

        def compute(meta_v, rows_v, sidx_v):
            def cpy(kk, _):
                sidx_v[pl.ds(kk * 16, 16)] = meta_v[pl.ds(CH + kk * 16, 16)]
                return 0
            lax.fori_loop(0, CH // 16, cpy, 0)

            def grp(e16, _):
                gbits = meta_v[pl.ds(2 * CH + e16 * 16, 16)]
                g16 = plsc.bitcast(gbits, jnp.float32)
                e0 = e16 * 16
                for j in range(16):
                    lane = jnp.full((16,), j, jnp.int32)
                    gb = lax.gather(
                        g16, lane[:, None],
                        lax.GatherDimensionNumbers(
                            offset_dims=(), collapsed_slice_dims=(0,),
                            start_index_map=(0,)),
                        (1,),
                        mode=lax.GatherScatterMode.PROMISE_IN_BOUNDS)
                    for kk in range(DH // 16):
                        sl = pl.ds(kk * 16, 16)
                        rows_v[e0 + j, sl] = rows_v[e0 + j, sl] * gb
                    rows_v[e0 + j, pl.ds(DH, 16)] = deg_vec
                return 0
            lax.fori_loop(0, CH // 16, grp, 0)

        def gather_start(meta_v, rows_v, gsem):
            pltpu.async_copy(
                table_hbm.at[meta_v.at[pl.ds(0, CH)]], rows_v, gsem)

        def gather_wait(meta_v, rows_v, gsem):
            pltpu.make_async_copy(
                table_hbm.at[meta_v.at[pl.ds(0, CH)]], rows_v, gsem).wait()

        def meta_start(c, meta_v, msem):
            pltpu.async_copy(meta_hbm.at[cid, rowbase + c], meta_v, msem)

        def meta_wait(c, meta_v, msem):
            pltpu.make_async_copy(
                meta_hbm.at[cid, rowbase + c], meta_v, msem).wait()

        def scatter_start(rows_v, sidx_v, ssem):
            pltpu.async_copy(rows_v, acc.at[sidx_v], ssem, add=True)

        def scatter_wait(rows_v, sidx_v, ssem):
            pltpu.make_async_copy(rows_v, acc.at[sidx_v], ssem).wait()

        meta_start(0, meta0, msem0)
        meta_wait(0, meta0, msem0)
        gather_start(meta0, rows0, gsem0)
        meta_start(1, meta1, msem1)

        def pair(i, _):
            meta_wait(2 * i + 1, meta1, msem1)

            @pl.when(i > 0)
            def _():
                scatter_wait(rows1, sidx1, ssem1)
            gather_start(meta1, rows1, gsem1)
            gather_wait(meta0, rows0, gsem0)
            compute(meta0, rows0, sidx0)
            scatter_start(rows0, sidx0, ssem0)

            @pl.when(i < NPAIR - 1)
            def _():
                meta_start(2 * i + 2, meta0, msem0)

            gather_wait(meta1, rows1, gsem1)
            compute(meta1, rows1, sidx1)
            scatter_wait(rows0, sidx0, ssem0)

            @pl.when(i < NPAIR - 1)
            def _():
                meta_wait(2 * i + 2, meta0, msem0)
                gather_start(meta0, rows0, gsem0)
            scatter_start(rows1, sidx1, ssem1)

            @pl.when(i < NPAIR - 1)
            def _():
                meta_start(2 * i + 3, meta1, msem1)
            return 0
        lax.fori_loop(0, NPAIR, pair, 0)

        scatter_wait(rows1, sidx1, ssem1)
        plsc.subcore_barrier()
        pltpu.sync_copy(acc.at[pl.ds(zbase, ROWS_PER_TILE)],
                        out_hbm.at[cid, pl.ds(zbase, ROWS_PER_TILE)])

    return k(table, meta)


def _epilogue(acc, x_dst, W_dst, b_dst, gamma, beta):
    BN = 400
    grid = N // BN

    def body(a_ref, xd_ref, wd_ref, bd_ref, gm_ref, bt_ref, out_ref):
        lo = a_ref[0, :, :DH]
        hi = a_ref[1, :, :DH]
        sums = jnp.concatenate([lo, hi], axis=1)
        deg = a_ref[0, :, DH:DH + 1]
        degc = jnp.maximum(deg, 1.0)
        z = lax.dot_general(xd_ref[...], wd_ref[...], (((1,), (1,)), ((), ())),
                            preferred_element_type=jnp.float32)
        pre = sums / degc + z + bd_ref[...]
        mu = jnp.mean(pre, axis=1, keepdims=True)
        var = jnp.mean(jnp.square(pre - mu), axis=1, keepdims=True)
        ln = (pre - mu) * lax.rsqrt(var + 1e-5) * gm_ref[...] + bt_ref[...]
        out_ref[...] = _gelu(ln)

    return pl.pallas_call(
        body,
        grid=(grid,),
        in_specs=[
            pl.BlockSpec((2, BN, W), lambda i: (0, i, 0)),
            pl.BlockSpec((BN, D), lambda i: (i, 0)),
            pl.BlockSpec((D, D), lambda i: (0, 0)),
            pl.BlockSpec((D,), lambda i: (0,)),
            pl.BlockSpec((D,), lambda i: (0,)),
            pl.BlockSpec((D,), lambda i: (0,)),
        ],
        out_specs=pl.BlockSpec((BN, D), lambda i: (i, 0)),
        out_shape=jax.ShapeDtypeStruct((N, D), jnp.float32),
    )(acc, x_dst, W_dst, b_dst, gamma, beta)


def kernel(x_src, x_dst, edge_index, edge_attr, W_src, W_dst, b_dst,
           w1, b1, w2, b2, gamma, beta):
    src = edge_index[0]
    dst = edge_index[1]
    pad = E_PAD - E

    g = _gates(edge_attr, w1, w2, b1, b2)

    src_pad = jnp.concatenate([src, jnp.zeros((pad,), jnp.int32)])
    dst_pad = jnp.concatenate([dst, jnp.full((pad,), N, jnp.int32)])
    g_pad = jnp.concatenate([g, jnp.zeros((pad,), jnp.float32)])
    gbits = lax.bitcast_convert_type(g_pad, jnp.int32).reshape(-1, CH)
    dst_r = dst_pad.reshape(-1, CH)

    def plane(offset):
        s = (src_pad + offset).reshape(-1, CH)
        return jnp.concatenate([s, dst_r, gbits], axis=1)
    meta = jnp.stack([plane(0), plane(N)])

    table = _build_table(x_src, W_src)
    acc = _sc_scatter(table, meta)
    return _epilogue(acc, x_dst, W_dst, b_dst, gamma, beta)


# speedup vs baseline: 3.2560x; 1.0046x over previous
import functools
import math

import jax
import jax.numpy as jnp
from jax import lax
from jax.experimental import pallas as pl
from jax.experimental.pallas import tpu as pltpu
from jax.experimental.pallas import tpu_sc as plsc

N = 10000
E = 160000
D = 256
ED = 16
DH = D // 2
W = 144
N_PAD = 10240
N_ACC = 10016
E_PAD = 163840
EPT = E_PAD // 16
CH = 128
NCHUNK = EPT // CH
NPAIR = NCHUNK // 2
MW = 3 * CH
ROWS_PER_TILE = N_ACC // 16

_SQRT2 = math.sqrt(2.0)


def _gelu(x):
    return 0.5 * x * (1.0 + lax.erf(x / _SQRT2))


def _table_body(x_ref, w_ref, out_ref):
    y = lax.dot_general(x_ref[...], w_ref[...], (((1,), (1,)), ((), ())),
                        preferred_element_type=jnp.float32)
    b = x_ref.shape[0]
    deg_pad = jnp.concatenate(
        [jnp.ones((b, 1), jnp.float32), jnp.zeros((b, W - DH - 1), jnp.float32)],
        axis=1)
    out_ref[0] = jnp.concatenate([y[:, :DH], deg_pad], axis=1)
    out_ref[1] = jnp.concatenate([y[:, DH:], deg_pad], axis=1)


def _build_table(x_src, W_src):
    BN = 400
    grid = N // BN
    out = pl.pallas_call(
        _table_body,
        grid=(grid,),
        in_specs=[
            pl.BlockSpec((BN, D), lambda i: (i, 0)),
            pl.BlockSpec((D, D), lambda i: (0, 0)),
        ],
        out_specs=pl.BlockSpec((2, BN, W), lambda i: (0, i, 0)),
        out_shape=jax.ShapeDtypeStruct((2, N, W), jnp.float32),
    )(x_src, W_src)
    return out.reshape(2 * N, W)


def _gates(edge_attr, w1, w2, b1, b2):
    BE = 1600
    grid = E // BE

    def body(e_ref, w1_ref, b1_ref, w2_ref, b2_ref, out_ref):
        h = lax.dot_general(e_ref[...], w1_ref[...], (((1,), (1,)), ((), ())),
                            preferred_element_type=jnp.float32)
        h = _gelu(h + b1_ref[...])
        g = jnp.sum(h * w2_ref[...], axis=1, keepdims=True)
        g = g + b2_ref[0]
        out_ref[...] = jax.nn.sigmoid(g).reshape(1, 8, BE // 8)

    out = pl.pallas_call(
        body,
        grid=(grid,),
        in_specs=[
            pl.BlockSpec((BE, ED), lambda i: (i, 0)),
            pl.BlockSpec((D, ED), lambda i: (0, 0)),
            pl.BlockSpec((D,), lambda i: (0,)),
            pl.BlockSpec((1, D), lambda i: (0, 0)),
            pl.BlockSpec((1,), lambda i: (0,)),
        ],
        out_specs=pl.BlockSpec((1, 8, BE // 8), lambda i: (i, 0, 0)),
        out_shape=jax.ShapeDtypeStruct((grid, 8, BE // 8), jnp.float32),
    )(edge_attr, w1, b1, w2, b2)
    return out.reshape(E)


def _sc_scatter(table, meta):
    mesh = plsc.VectorSubcoreMesh(core_axis_name="c", subcore_axis_name="s")

    @functools.partial(
        pl.kernel,
        mesh=mesh,
        out_type=jax.ShapeDtypeStruct((2, N_PAD, W), jnp.float32),
        scratch_types=[
            pltpu.VMEM((CH, W), jnp.float32),
            pltpu.VMEM((CH, W), jnp.float32),
            pltpu.VMEM((MW,), jnp.int32),
            pltpu.VMEM((MW,), jnp.int32),
            pltpu.VMEM((CH,), jnp.int32),
            pltpu.VMEM((CH,), jnp.int32),
            pltpu.VMEM_SHARED((N_ACC, W), jnp.float32),
            pltpu.SemaphoreType.DMA,
            pltpu.SemaphoreType.DMA,
            pltpu.SemaphoreType.DMA,
            pltpu.SemaphoreType.DMA,
            pltpu.SemaphoreType.DMA,
            pltpu.SemaphoreType.DMA,
        ],
        compiler_params=pltpu.CompilerParams(use_tc_tiling_on_sc=False,
                                             needs_layout_passes=False),
    )
    def k(table_hbm, meta_hbm, out_hbm,
          rows0, rows1, meta0, meta1, sidx0, sidx1, acc,
          gsem0, gsem1, msem0, msem1, ssem0, ssem1):
        cid = lax.axis_index("c")
        sid = lax.axis_index("s")
        rowbase = sid * NCHUNK

        def zrow(i, _):
            r = i // (W // 16)
            c = (i % (W // 16)) * 16
            rows0[r, pl.ds(c, 16)] = jnp.zeros((16,), jnp.float32)
            return 0
        lax.fori_loop(0, CH * (W // 16), zrow, 0)
        zbase = sid * ROWS_PER_TILE
        for j in range(ROWS_PER_TILE // CH):
            pltpu.sync_copy(rows0, acc.at[pl.ds(zbase + j * CH, CH)])
        rem = ROWS_PER_TILE % CH
        if rem:
            pltpu.sync_copy(
                rows0.at[pl.ds(0, rem)],
                acc.at[pl.ds(zbase + (ROWS_PER_TILE // CH) * CH, rem)])
        plsc.subcore_barrier()

        def compute(meta_v, rows_v, sidx_v):
            @plsc.parallel_loop(0, CH // 16)
            def cpy(kk):
                sidx_v[pl.ds(kk * 16, 16)] = meta_v[pl.ds(CH + kk * 16, 16)]

            @plsc.parallel_loop(0, CH // 16)
            def grp(e16):
                gbits = meta_v[pl.ds(2 * CH + e16 * 16, 16)]
                g16 = plsc.bitcast(gbits, jnp.float32)
                e0 = e16 * 16
                for j in range(16):
                    lane = jnp.full((16,), j, jnp.int32)
                    gb = lax.gather(
                        g16, lane[:, None],
                        lax.GatherDimensionNumbers(
                            offset_dims=(), collapsed_slice_dims=(0,),
                            start_index_map=(0,)),
                        (1,),
                        mode=lax.GatherScatterMode.PROMISE_IN_BOUNDS)
                    for kk in range(DH // 16):
                        sl = pl.ds(kk * 16, 16)
                        rows_v[e0 + j, sl] = rows_v[e0 + j, sl] * gb

        def gather_start(meta_v, rows_v, gsem):
            pltpu.async_copy(
                table_hbm.at[meta_v.at[pl.ds(0, CH)]], rows_v, gsem)

        def gather_wait(meta_v, rows_v, gsem):
            pltpu.make_async_copy(
                table_hbm.at[meta_v.at[pl.ds(0, CH)]], rows_v, gsem).wait()

        def meta_start(c, meta_v, msem):
            pltpu.async_copy(meta_hbm.at[cid, rowbase + c], meta_v, msem)

        def meta_wait(c, meta_v, msem):
            pltpu.make_async_copy(
                meta_hbm.at[cid, rowbase + c], meta_v, msem).wait()

        def scatter_start(rows_v, sidx_v, ssem):
            pltpu.async_copy(rows_v, acc.at[sidx_v], ssem, add=True)

        def scatter_wait(rows_v, sidx_v, ssem):
            pltpu.make_async_copy(rows_v, acc.at[sidx_v], ssem).wait()

        meta_start(0, meta0, msem0)
        meta_wait(0, meta0, msem0)
        gather_start(meta0, rows0, gsem0)
        meta_start(1, meta1, msem1)

        def pair(i, _):
            meta_wait(2 * i + 1, meta1, msem1)

            @pl.when(i > 0)
            def _():
                scatter_wait(rows1, sidx1, ssem1)
            gather_start(meta1, rows1, gsem1)
            gather_wait(meta0, rows0, gsem0)
            compute(meta0, rows0, sidx0)
            scatter_start(rows0, sidx0, ssem0)

            @pl.when(i < NPAIR - 1)
            def _():
                meta_start(2 * i + 2, meta0, msem0)

            gather_wait(meta1, rows1, gsem1)
            compute(meta1, rows1, sidx1)
            scatter_wait(rows0, sidx0, ssem0)

            @pl.when(i < NPAIR - 1)
            def _():
                meta_wait(2 * i + 2, meta0, msem0)
                gather_start(meta0, rows0, gsem0)
            scatter_start(rows1, sidx1, ssem1)

            @pl.when(i < NPAIR - 1)
            def _():
                meta_start(2 * i + 3, meta1, msem1)
            return 0
        lax.fori_loop(0, NPAIR, pair, 0)

        scatter_wait(rows1, sidx1, ssem1)
        plsc.subcore_barrier()
        pltpu.sync_copy(acc.at[pl.ds(zbase, ROWS_PER_TILE)],
                        out_hbm.at[cid, pl.ds(zbase, ROWS_PER_TILE)])

    return k(table, meta)


def _epilogue(acc, x_dst, W_dst, b_dst, gamma, beta):
    BN = 400
    grid = N // BN

    def body(a_ref, xd_ref, wd_ref, bd_ref, gm_ref, bt_ref, out_ref):
        lo = a_ref[0, :, :DH]
        hi = a_ref[1, :, :DH]
        sums = jnp.concatenate([lo, hi], axis=1)
        deg = a_ref[0, :, DH:DH + 1]
        degc = jnp.maximum(deg, 1.0)
        z = lax.dot_general(xd_ref[...], wd_ref[...], (((1,), (1,)), ((), ())),
                            preferred_element_type=jnp.float32)
        pre = sums / degc + z + bd_ref[...]
        mu = jnp.mean(pre, axis=1, keepdims=True)
        var = jnp.mean(jnp.square(pre - mu), axis=1, keepdims=True)
        ln = (pre - mu) * lax.rsqrt(var + 1e-5) * gm_ref[...] + bt_ref[...]
        out_ref[...] = _gelu(ln)

    return pl.pallas_call(
        body,
        grid=(grid,),
        in_specs=[
            pl.BlockSpec((2, BN, W), lambda i: (0, i, 0)),
            pl.BlockSpec((BN, D), lambda i: (i, 0)),
            pl.BlockSpec((D, D), lambda i: (0, 0)),
            pl.BlockSpec((D,), lambda i: (0,)),
            pl.BlockSpec((D,), lambda i: (0,)),
            pl.BlockSpec((D,), lambda i: (0,)),
        ],
        out_specs=pl.BlockSpec((BN, D), lambda i: (i, 0)),
        out_shape=jax.ShapeDtypeStruct((N, D), jnp.float32),
    )(acc, x_dst, W_dst, b_dst, gamma, beta)


def kernel(x_src, x_dst, edge_index, edge_attr, W_src, W_dst, b_dst,
           w1, b1, w2, b2, gamma, beta):
    src = edge_index[0]
    dst = edge_index[1]
    pad = E_PAD - E

    g = _gates(edge_attr, w1, w2, b1, b2)

    src_pad = jnp.concatenate([src, jnp.zeros((pad,), jnp.int32)])
    dst_pad = jnp.concatenate([dst, jnp.full((pad,), N, jnp.int32)])
    g_pad = jnp.concatenate([g, jnp.zeros((pad,), jnp.float32)])
    gbits = lax.bitcast_convert_type(g_pad, jnp.int32).reshape(-1, CH)
    dst_r = dst_pad.reshape(-1, CH)

    def plane(offset):
        s = (src_pad + offset).reshape(-1, CH)
        return jnp.concatenate([s, dst_r, gbits], axis=1)
    meta = jnp.stack([plane(0), plane(N)])

    table = _build_table(x_src, W_src)
    acc = _sc_scatter(table, meta)
    return _epilogue(acc, x_dst, W_dst, b_dst, gamma, beta)


# speedup vs baseline: 3.3392x; 1.0256x over previous
import functools
import math

import jax
import jax.numpy as jnp
from jax import lax
from jax.experimental import pallas as pl
from jax.experimental.pallas import tpu as pltpu
from jax.experimental.pallas import tpu_sc as plsc

N = 10000
E = 160000
D = 256
ED = 16
DH = D // 2
W = 144
N_PAD = 10240
N_ACC = 10016
E_PAD = 163840
EPT = E_PAD // 16
CH = 64
NCHUNK = EPT // CH
NBUF = 4
MW = 3 * CH
ROWS_PER_TILE = N_ACC // 16

_SQRT2 = math.sqrt(2.0)


def _gelu(x):
    return 0.5 * x * (1.0 + lax.erf(x / _SQRT2))


def _table_body(x_ref, w_ref, out_ref):
    y = lax.dot_general(x_ref[...], w_ref[...], (((1,), (1,)), ((), ())),
                        preferred_element_type=jnp.float32)
    b = x_ref.shape[0]
    deg_pad = jnp.concatenate(
        [jnp.ones((b, 1), jnp.float32), jnp.zeros((b, W - DH - 1), jnp.float32)],
        axis=1)
    out_ref[0] = jnp.concatenate([y[:, :DH], deg_pad], axis=1)
    out_ref[1] = jnp.concatenate([y[:, DH:], deg_pad], axis=1)


def _build_table(x_src, W_src):
    BN = 400
    grid = N // BN
    out = pl.pallas_call(
        _table_body,
        grid=(grid,),
        in_specs=[
            pl.BlockSpec((BN, D), lambda i: (i, 0)),
            pl.BlockSpec((D, D), lambda i: (0, 0)),
        ],
        out_specs=pl.BlockSpec((2, BN, W), lambda i: (0, i, 0)),
        out_shape=jax.ShapeDtypeStruct((2, N, W), jnp.float32),
    )(x_src, W_src)
    return out.reshape(2 * N, W)


def _gates(edge_attr, w1, w2, b1, b2):
    BE = 1600
    grid = E // BE

    def body(e_ref, w1_ref, b1_ref, w2_ref, b2_ref, out_ref):
        h = lax.dot_general(e_ref[...], w1_ref[...], (((1,), (1,)), ((), ())),
                            preferred_element_type=jnp.float32)
        h = _gelu(h + b1_ref[...])
        g = jnp.sum(h * w2_ref[...], axis=1, keepdims=True)
        g = g + b2_ref[0]
        out_ref[...] = jax.nn.sigmoid(g).reshape(1, 8, BE // 8)

    out = pl.pallas_call(
        body,
        grid=(grid,),
        in_specs=[
            pl.BlockSpec((BE, ED), lambda i: (i, 0)),
            pl.BlockSpec((D, ED), lambda i: (0, 0)),
            pl.BlockSpec((D,), lambda i: (0,)),
            pl.BlockSpec((1, D), lambda i: (0, 0)),
            pl.BlockSpec((1,), lambda i: (0,)),
        ],
        out_specs=pl.BlockSpec((1, 8, BE // 8), lambda i: (i, 0, 0)),
        out_shape=jax.ShapeDtypeStruct((grid, 8, BE // 8), jnp.float32),
    )(edge_attr, w1, b1, w2, b2)
    return out.reshape(E)


def _sc_scatter(table, meta):
    mesh = plsc.VectorSubcoreMesh(core_axis_name="c", subcore_axis_name="s")

    @functools.partial(
        pl.kernel,
        mesh=mesh,
        out_type=jax.ShapeDtypeStruct((2, N_PAD, W), jnp.float32),
        scratch_types=(
            [pltpu.VMEM((CH, W), jnp.float32) for _ in range(NBUF)]
            + [pltpu.VMEM((MW,), jnp.int32) for _ in range(NBUF)]
            + [pltpu.VMEM((CH,), jnp.int32) for _ in range(NBUF)]
            + [pltpu.VMEM_SHARED((N_ACC, W), jnp.float32)]
            + [pltpu.SemaphoreType.DMA for _ in range(3 * NBUF)]
        ),
        compiler_params=pltpu.CompilerParams(use_tc_tiling_on_sc=False,
                                             needs_layout_passes=False),
    )
    def k(table_hbm, meta_hbm, out_hbm, *refs):
        rows = refs[0:NBUF]
        meta = refs[NBUF:2 * NBUF]
        sidx = refs[2 * NBUF:3 * NBUF]
        acc = refs[3 * NBUF]
        gsem = refs[3 * NBUF + 1:3 * NBUF + 1 + NBUF]
        msem = refs[3 * NBUF + 1 + NBUF:3 * NBUF + 1 + 2 * NBUF]
        ssem = refs[3 * NBUF + 1 + 2 * NBUF:3 * NBUF + 1 + 3 * NBUF]
        cid = lax.axis_index("c")
        sid = lax.axis_index("s")
        rowbase = sid * NCHUNK

        def zrow(i, _):
            r = i // (W // 16)
            c = (i % (W // 16)) * 16
            rows[0][r, pl.ds(c, 16)] = jnp.zeros((16,), jnp.float32)
            return 0
        lax.fori_loop(0, CH * (W // 16), zrow, 0)
        zbase = sid * ROWS_PER_TILE
        for j in range(ROWS_PER_TILE // CH):
            pltpu.sync_copy(rows[0], acc.at[pl.ds(zbase + j * CH, CH)])
        rem = ROWS_PER_TILE % CH
        if rem:
            pltpu.sync_copy(
                rows[0].at[pl.ds(0, rem)],
                acc.at[pl.ds(zbase + (ROWS_PER_TILE // CH) * CH, rem)])
        plsc.subcore_barrier()

        def compute(meta_v, rows_v, sidx_v):
            @plsc.parallel_loop(0, CH // 16)
            def cpy(kk):
                sidx_v[pl.ds(kk * 16, 16)] = meta_v[pl.ds(CH + kk * 16, 16)]

            @plsc.parallel_loop(0, CH // 16)
            def grp(e16):
                gbits = meta_v[pl.ds(2 * CH + e16 * 16, 16)]
                g16 = plsc.bitcast(gbits, jnp.float32)
                e0 = e16 * 16
                for j in range(16):
                    lane = jnp.full((16,), j, jnp.int32)
                    gb = lax.gather(
                        g16, lane[:, None],
                        lax.GatherDimensionNumbers(
                            offset_dims=(), collapsed_slice_dims=(0,),
                            start_index_map=(0,)),
                        (1,),
                        mode=lax.GatherScatterMode.PROMISE_IN_BOUNDS)
                    for kk in range(DH // 16):
                        sl = pl.ds(kk * 16, 16)
                        rows_v[e0 + j, sl] = rows_v[e0 + j, sl] * gb

        def gather_start(b):
            pltpu.async_copy(
                table_hbm.at[meta[b].at[pl.ds(0, CH)]], rows[b], gsem[b])

        def gather_wait(b):
            pltpu.make_async_copy(
                table_hbm.at[meta[b].at[pl.ds(0, CH)]], rows[b],
                gsem[b]).wait()

        def meta_start(c, b):
            pltpu.async_copy(meta_hbm.at[cid, rowbase + c], meta[b], msem[b])

        def meta_wait(c, b):
            pltpu.make_async_copy(
                meta_hbm.at[cid, rowbase + c], meta[b], msem[b]).wait()

        def scatter_start(b):
            pltpu.async_copy(rows[b], acc.at[sidx[b]], ssem[b], add=True)

        def scatter_wait(b):
            pltpu.make_async_copy(rows[b], acc.at[sidx[b]], ssem[b]).wait()

        meta_start(0, 0)
        meta_start(1, 1)
        meta_wait(0, 0)
        gather_start(0)
        meta_wait(1, 1)
        gather_start(1)
        meta_start(2, 2)
        meta_start(3, 3)

        def step(c, b):
            gather_wait(b)
            compute(meta[b], rows[b], sidx[b])
            scatter_start(b)
            b2 = (b + 2) % NBUF

            @pl.when(c >= 2)
            def _():
                scatter_wait(b2)

            @pl.when(c + 2 < NCHUNK)
            def _():
                meta_wait(c + 2, b2)
                gather_start(b2)

            @pl.when(c + 4 < NCHUNK)
            def _():
                meta_start(c + 4, b)

        def group(i, _):
            for bb in range(NBUF):
                step(i * NBUF + bb, bb)
            return 0
        lax.fori_loop(0, NCHUNK // NBUF, group, 0)

        scatter_wait((NCHUNK - 2) % NBUF)
        scatter_wait((NCHUNK - 1) % NBUF)
        plsc.subcore_barrier()
        pltpu.sync_copy(acc.at[pl.ds(zbase, ROWS_PER_TILE)],
                        out_hbm.at[cid, pl.ds(zbase, ROWS_PER_TILE)])

    return k(table, meta)


def _epilogue(acc, x_dst, W_dst, b_dst, gamma, beta):
    BN = 400
    grid = N // BN

    def body(a_ref, xd_ref, wd_ref, bd_ref, gm_ref, bt_ref, out_ref):
        lo = a_ref[0, :, :DH]
        hi = a_ref[1, :, :DH]
        sums = jnp.concatenate([lo, hi], axis=1)
        deg = a_ref[0, :, DH:DH + 1]
        degc = jnp.maximum(deg, 1.0)
        z = lax.dot_general(xd_ref[...], wd_ref[...], (((1,), (1,)), ((), ())),
                            preferred_element_type=jnp.float32)
        pre = sums / degc + z + bd_ref[...]
        mu = jnp.mean(pre, axis=1, keepdims=True)
        var = jnp.mean(jnp.square(pre - mu), axis=1, keepdims=True)
        ln = (pre - mu) * lax.rsqrt(var + 1e-5) * gm_ref[...] + bt_ref[...]
        out_ref[...] = _gelu(ln)

    return pl.pallas_call(
        body,
        grid=(grid,),
        in_specs=[
            pl.BlockSpec((2, BN, W), lambda i: (0, i, 0)),
            pl.BlockSpec((BN, D), lambda i: (i, 0)),
            pl.BlockSpec((D, D), lambda i: (0, 0)),
            pl.BlockSpec((D,), lambda i: (0,)),
            pl.BlockSpec((D,), lambda i: (0,)),
            pl.BlockSpec((D,), lambda i: (0,)),
        ],
        out_specs=pl.BlockSpec((BN, D), lambda i: (i, 0)),
        out_shape=jax.ShapeDtypeStruct((N, D), jnp.float32),
    )(acc, x_dst, W_dst, b_dst, gamma, beta)


def kernel(x_src, x_dst, edge_index, edge_attr, W_src, W_dst, b_dst,
           w1, b1, w2, b2, gamma, beta):
    src = edge_index[0]
    dst = edge_index[1]
    pad = E_PAD - E

    g = _gates(edge_attr, w1, w2, b1, b2)

    src_pad = jnp.concatenate([src, jnp.zeros((pad,), jnp.int32)])
    dst_pad = jnp.concatenate([dst, jnp.full((pad,), N, jnp.int32)])
    g_pad = jnp.concatenate([g, jnp.zeros((pad,), jnp.float32)])
    gbits = lax.bitcast_convert_type(g_pad, jnp.int32).reshape(-1, CH)
    dst_r = dst_pad.reshape(-1, CH)

    def plane(offset):
        s = (src_pad + offset).reshape(-1, CH)
        return jnp.concatenate([s, dst_r, gbits], axis=1)
    meta = jnp.stack([plane(0), plane(N)])

    table = _build_table(x_src, W_src)
    acc = _sc_scatter(table, meta)
    return _epilogue(acc, x_dst, W_dst, b_dst, gamma, beta)


# speedup vs baseline: 3.5916x; 1.0756x over previous
import functools
import math

import jax
import jax.numpy as jnp
from jax import lax
from jax.experimental import pallas as pl
from jax.experimental.pallas import tpu as pltpu
from jax.experimental.pallas import tpu_sc as plsc

N = 10000
E = 160000
D = 256
ED = 16
DH = D // 2
W = 160
N_PAD = 10240
N_ACC = 10016
E_PAD = 163840
EPT = E_PAD // 16
CH = 128
NCHUNK = EPT // CH
NBUF = 4
MW = 3 * CH
ROWS_PER_TILE = N_ACC // 16

_SQRT2 = math.sqrt(2.0)


def _gelu(x):
    return 0.5 * x * (1.0 + lax.erf(x / _SQRT2))


def _table_body(x_ref, w_ref, out_ref):
    y = lax.dot_general(x_ref[...], w_ref[...], (((1,), (1,)), ((), ())),
                        preferred_element_type=jnp.float32)
    b = x_ref.shape[0]
    deg_pad = jnp.concatenate(
        [jnp.ones((b, 1), jnp.float32), jnp.zeros((b, W - DH - 1), jnp.float32)],
        axis=1)
    out_ref[0] = jnp.concatenate(
        [y[:, :DH], deg_pad], axis=1).astype(jnp.bfloat16)
    out_ref[1] = jnp.concatenate(
        [y[:, DH:], deg_pad], axis=1).astype(jnp.bfloat16)


def _build_table(x_src, W_src):
    BN = 400
    grid = N // BN
    out = pl.pallas_call(
        _table_body,
        grid=(grid,),
        in_specs=[
            pl.BlockSpec((BN, D), lambda i: (i, 0)),
            pl.BlockSpec((D, D), lambda i: (0, 0)),
        ],
        out_specs=pl.BlockSpec((2, BN, W), lambda i: (0, i, 0)),
        out_shape=jax.ShapeDtypeStruct((2, N, W), jnp.bfloat16),
    )(x_src, W_src)
    return out.reshape(2 * N, W)


def _gates(edge_attr, w1, w2, b1, b2):
    BE = 1600
    grid = E // BE

    def body(e_ref, w1_ref, b1_ref, w2_ref, b2_ref, out_ref):
        h = lax.dot_general(e_ref[...], w1_ref[...], (((1,), (1,)), ((), ())),
                            preferred_element_type=jnp.float32)
        h = _gelu(h + b1_ref[...])
        g = jnp.sum(h * w2_ref[...], axis=1, keepdims=True)
        g = g + b2_ref[0]
        out_ref[...] = jax.nn.sigmoid(g).reshape(1, 8, BE // 8)

    out = pl.pallas_call(
        body,
        grid=(grid,),
        in_specs=[
            pl.BlockSpec((BE, ED), lambda i: (i, 0)),
            pl.BlockSpec((D, ED), lambda i: (0, 0)),
            pl.BlockSpec((D,), lambda i: (0,)),
            pl.BlockSpec((1, D), lambda i: (0, 0)),
            pl.BlockSpec((1,), lambda i: (0,)),
        ],
        out_specs=pl.BlockSpec((1, 8, BE // 8), lambda i: (i, 0, 0)),
        out_shape=jax.ShapeDtypeStruct((grid, 8, BE // 8), jnp.float32),
    )(edge_attr, w1, b1, w2, b2)
    return out.reshape(E)


def _sc_scatter(table, meta):
    mesh = plsc.VectorSubcoreMesh(core_axis_name="c", subcore_axis_name="s")

    @functools.partial(
        pl.kernel,
        mesh=mesh,
        out_type=jax.ShapeDtypeStruct((2, N_PAD, W), jnp.bfloat16),
        scratch_types=(
            [pltpu.VMEM((CH, W), jnp.bfloat16) for _ in range(NBUF)]
            + [pltpu.VMEM((MW,), jnp.int32) for _ in range(NBUF)]
            + [pltpu.VMEM((CH,), jnp.int32) for _ in range(NBUF)]
            + [pltpu.VMEM_SHARED((N_ACC, W), jnp.bfloat16)]
            + [pltpu.SemaphoreType.DMA for _ in range(3 * NBUF)]
        ),
        compiler_params=pltpu.CompilerParams(use_tc_tiling_on_sc=False,
                                             needs_layout_passes=False),
    )
    def k(table_hbm, meta_hbm, out_hbm, *refs):
        rows = refs[0:NBUF]
        meta = refs[NBUF:2 * NBUF]
        sidx = refs[2 * NBUF:3 * NBUF]
        acc = refs[3 * NBUF]
        gsem = refs[3 * NBUF + 1:3 * NBUF + 1 + NBUF]
        msem = refs[3 * NBUF + 1 + NBUF:3 * NBUF + 1 + 2 * NBUF]
        ssem = refs[3 * NBUF + 1 + 2 * NBUF:3 * NBUF + 1 + 3 * NBUF]
        cid = lax.axis_index("c")
        sid = lax.axis_index("s")
        rowbase = sid * NCHUNK

        def zrow(i, _):
            r = i // (W // 32)
            c = (i % (W // 32)) * 32
            rows[0][r, pl.ds(c, 32)] = jnp.zeros((32,), jnp.bfloat16)
            return 0
        lax.fori_loop(0, CH * (W // 32), zrow, 0)
        zbase = sid * ROWS_PER_TILE
        for j in range(ROWS_PER_TILE // CH):
            pltpu.sync_copy(rows[0], acc.at[pl.ds(zbase + j * CH, CH)])
        rem = ROWS_PER_TILE % CH
        if rem:
            pltpu.sync_copy(
                rows[0].at[pl.ds(0, rem)],
                acc.at[pl.ds(zbase + (ROWS_PER_TILE // CH) * CH, rem)])
        plsc.subcore_barrier()

        def compute(meta_v, rows_v, sidx_v):
            @plsc.parallel_loop(0, CH // 16)
            def cpy(kk):
                sidx_v[pl.ds(kk * 16, 16)] = meta_v[pl.ds(CH + kk * 16, 16)]

            @plsc.parallel_loop(0, CH // 16)
            def grp(e16):
                gbits = meta_v[pl.ds(2 * CH + e16 * 16, 16)]
                g16 = plsc.bitcast(gbits, jnp.float32)
                e0 = e16 * 16
                for j in range(16):
                    lane = jnp.full((16,), j, jnp.int32)
                    gb = lax.gather(
                        g16, lane[:, None],
                        lax.GatherDimensionNumbers(
                            offset_dims=(), collapsed_slice_dims=(0,),
                            start_index_map=(0,)),
                        (1,),
                        mode=lax.GatherScatterMode.PROMISE_IN_BOUNDS)
                    gb2 = plsc.pack(gb, gb, format=plsc.PackFormat.INTERLEAVED)
                    for kk in range(DH // 32):
                        sl = pl.ds(kk * 32, 32)
                        rows_v[e0 + j, sl] = rows_v[e0 + j, sl] * gb2

        def gather_start(b):
            pltpu.async_copy(
                table_hbm.at[meta[b].at[pl.ds(0, CH)]], rows[b], gsem[b])

        def gather_wait(b):
            pltpu.make_async_copy(
                table_hbm.at[meta[b].at[pl.ds(0, CH)]], rows[b],
                gsem[b]).wait()

        def meta_start(c, b):
            pltpu.async_copy(meta_hbm.at[cid, rowbase + c], meta[b], msem[b])

        def meta_wait(c, b):
            pltpu.make_async_copy(
                meta_hbm.at[cid, rowbase + c], meta[b], msem[b]).wait()

        def scatter_start(b):
            pltpu.async_copy(rows[b], acc.at[sidx[b]], ssem[b], add=True)

        def scatter_wait(b):
            pltpu.make_async_copy(rows[b], acc.at[sidx[b]], ssem[b]).wait()

        meta_start(0, 0)
        meta_start(1, 1)
        meta_wait(0, 0)
        gather_start(0)
        meta_wait(1, 1)
        gather_start(1)
        meta_start(2, 2)
        meta_start(3, 3)

        def step(c, b):
            gather_wait(b)
            compute(meta[b], rows[b], sidx[b])
            scatter_start(b)
            b2 = (b + 2) % NBUF

            @pl.when(c >= 2)
            def _():
                scatter_wait(b2)

            @pl.when(c + 2 < NCHUNK)
            def _():
                meta_wait(c + 2, b2)
                gather_start(b2)

            @pl.when(c + 4 < NCHUNK)
            def _():
                meta_start(c + 4, b)

        def group(i, _):
            for bb in range(NBUF):
                step(i * NBUF + bb, bb)
            return 0
        lax.fori_loop(0, NCHUNK // NBUF, group, 0)

        scatter_wait((NCHUNK - 2) % NBUF)
        scatter_wait((NCHUNK - 1) % NBUF)
        plsc.subcore_barrier()
        pltpu.sync_copy(acc.at[pl.ds(zbase, ROWS_PER_TILE)],
                        out_hbm.at[cid, pl.ds(zbase, ROWS_PER_TILE)])

    return k(table, meta)


def _epilogue(acc, x_dst, W_dst, b_dst, gamma, beta):
    BN = 400
    grid = N // BN

    def body(a_ref, xd_ref, wd_ref, bd_ref, gm_ref, bt_ref, out_ref):
        lo = a_ref[0, :, :DH].astype(jnp.float32)
        hi = a_ref[1, :, :DH].astype(jnp.float32)
        sums = jnp.concatenate([lo, hi], axis=1)
        deg = a_ref[0, :, DH:DH + 1].astype(jnp.float32)
        degc = jnp.maximum(deg, 1.0)
        z = lax.dot_general(xd_ref[...], wd_ref[...], (((1,), (1,)), ((), ())),
                            preferred_element_type=jnp.float32)
        pre = sums / degc + z + bd_ref[...]
        mu = jnp.mean(pre, axis=1, keepdims=True)
        var = jnp.mean(jnp.square(pre - mu), axis=1, keepdims=True)
        ln = (pre - mu) * lax.rsqrt(var + 1e-5) * gm_ref[...] + bt_ref[...]
        out_ref[...] = _gelu(ln)

    return pl.pallas_call(
        body,
        grid=(grid,),
        in_specs=[
            pl.BlockSpec((2, BN, W), lambda i: (0, i, 0)),
            pl.BlockSpec((BN, D), lambda i: (i, 0)),
            pl.BlockSpec((D, D), lambda i: (0, 0)),
            pl.BlockSpec((D,), lambda i: (0,)),
            pl.BlockSpec((D,), lambda i: (0,)),
            pl.BlockSpec((D,), lambda i: (0,)),
        ],
        out_specs=pl.BlockSpec((BN, D), lambda i: (i, 0)),
        out_shape=jax.ShapeDtypeStruct((N, D), jnp.float32),
    )(acc, x_dst, W_dst, b_dst, gamma, beta)


def kernel(x_src, x_dst, edge_index, edge_attr, W_src, W_dst, b_dst,
           w1, b1, w2, b2, gamma, beta):
    src = edge_index[0]
    dst = edge_index[1]
    pad = E_PAD - E

    g = _gates(edge_attr, w1, w2, b1, b2)

    src_pad = jnp.concatenate([src, jnp.zeros((pad,), jnp.int32)])
    dst_pad = jnp.concatenate([dst, jnp.full((pad,), N, jnp.int32)])
    g_pad = jnp.concatenate([g, jnp.zeros((pad,), jnp.float32)])
    gbits = lax.bitcast_convert_type(g_pad, jnp.int32).reshape(-1, CH)
    dst_r = dst_pad.reshape(-1, CH)

    def plane(offset):
        s = (src_pad + offset).reshape(-1, CH)
        return jnp.concatenate([s, dst_r, gbits], axis=1)
    meta = jnp.stack([plane(0), plane(N)])

    table = _build_table(x_src, W_src)
    acc = _sc_scatter(table, meta)
    return _epilogue(acc, x_dst, W_dst, b_dst, gamma, beta)
